# baseline (speedup 1.0000x reference)
import jax
import jax.numpy as jnp
from jax.experimental import pallas as pl

NUM_NODES = 10000
RNN_UNITS = 32
OUTPUT_DIM = 1
MAX_DIFF = 2
NUM_MATRICES = 5


def _spmm(in_idx, out_idx, w, x):
    return jnp.zeros((NUM_NODES, x.shape[1]), x.dtype).at[out_idx].add(w[:, None] * x[in_idx])


def _gconv(supports, inputs, state, W, b, in_dim):
    B = inputs.shape[0]
    x = jnp.concatenate([inputs.reshape(B, NUM_NODES, in_dim), state.reshape(B, NUM_NODES, RNN_UNITS)], axis=2)
    isz = in_dim + RNN_UNITS
    x0 = jnp.transpose(x, (1, 2, 0)).reshape(NUM_NODES, isz * B)
    xs = [x0]
    for (in_idx, out_idx, w) in supports:
        x1 = _spmm(in_idx, out_idx, w, x0)
        xs.append(x1)
        for k in range(2, MAX_DIFF + 1):
            x2 = 2.0 * _spmm(in_idx, out_idx, w, x1) - x0
            xs.append(x2)
            x1, x0 = x2, x1
    xcat = jnp.stack(xs, axis=0).reshape(NUM_MATRICES, NUM_NODES, isz, B)
    xcat = jnp.transpose(xcat, (3, 1, 2, 0)).reshape(B * NUM_NODES, isz * NUM_MATRICES)
    return xcat @ W + b


def _dcgru_cell(supports, inputs, hx, Wg, bg, Wc, bc, in_dim):
    value = jax.nn.sigmoid(_gconv(supports, inputs, hx, Wg, bg, in_dim))
    value = value.reshape(-1, NUM_NODES, 2 * RNN_UNITS)
    r, u = jnp.split(value, 2, axis=-1)
    r = r.reshape(-1, NUM_NODES * RNN_UNITS)
    u = u.reshape(-1, NUM_NODES * RNN_UNITS)
    c = _gconv(supports, inputs, r * hx, Wc, bc, in_dim)
    c = jnp.tanh(c.reshape(-1, NUM_NODES * RNN_UNITS))
    return u * hx + (1.0 - u) * c


def _proj_body(x_ref, w_ref, b_ref, o_ref):
    o_ref[...] = x_ref[...] @ w_ref[...] + b_ref[...]


def _project(output, Wp, bp):
    x = output.reshape(-1, RNN_UNITS)
    M = x.shape[0]
    BLK = 8000
    return pl.pallas_call(
        _proj_body,
        grid=(M // BLK,),
        in_specs=[
            pl.BlockSpec((BLK, RNN_UNITS), lambda i: (i, 0)),
            pl.BlockSpec((RNN_UNITS, OUTPUT_DIM), lambda i: (0, 0)),
            pl.BlockSpec((OUTPUT_DIM,), lambda i: (0,)),
        ],
        out_specs=pl.BlockSpec((BLK, OUTPUT_DIM), lambda i: (i, 0)),
        out_shape=jax.ShapeDtypeStruct((M, OUTPUT_DIM), jnp.float32),
    )(x, Wp, bp)


def kernel(inputs, hidden_state, edge_index, edge_weight, Wg0, bg0, Wc0, bc0, Wg1, bg1, Wc1, bc1, Wp, bp):
    src = edge_index[0]
    dst = edge_index[1]
    deg_out = jnp.zeros((NUM_NODES,), edge_weight.dtype).at[src].add(edge_weight)
    deg_in = jnp.zeros((NUM_NODES,), edge_weight.dtype).at[dst].add(edge_weight)
    w1 = edge_weight / deg_out[src]
    w2 = edge_weight / deg_in[dst]
    supports = [(src, dst, w1), (dst, src, w2)]
    params = [(Wg0, bg0, Wc0, bc0, OUTPUT_DIM), (Wg1, bg1, Wc1, bc1, RNN_UNITS)]
    output = inputs
    hidden_states = []
    for layer_num, (Wg, bg, Wc, bc, in_dim) in enumerate(params):
        next_hidden = _dcgru_cell(supports, output, hidden_state[layer_num], Wg, bg, Wc, bc, in_dim)
        hidden_states.append(next_hidden)
        output = next_hidden
    projected = _project(output, Wp, bp)
    output = projected.reshape(-1, NUM_NODES * OUTPUT_DIM)
    return (output, jnp.stack(hidden_states))


# speedup vs baseline: 1.1427x; 1.1427x over previous
import functools

import jax
import jax.numpy as jnp
from jax import lax
from jax.experimental import pallas as pl
from jax.experimental.pallas import tpu as pltpu
from jax.experimental.pallas import tpu_sc as plsc

N = 10000
NP = 10240
E = 160000
EP = 163840
B = 16
RNN = 32
TILES = 32
NPT = NP // TILES
SR = 64
NSUB = NPT // SR
ESTG = 1024
EBLK = 128
NBLK = 256


def _sc_mesh():
    return plsc.VectorSubcoreMesh(core_axis_name="c", subcore_axis_name="s")


@functools.cache
def _spmm_kernel(nc, wc, cheby):
    nchunk = wc // 16

    def body(x_hbm, aux_hbm, gidx_hbm, sidx_hbm, w_hbm, meta_hbm, y_hbm,
             meta_v, gidx_v, sidx_v, w_v, rows_v, acc_v):
        t = lax.axis_index("s") * 2 + lax.axis_index("c")
        pltpu.sync_copy(meta_hbm, meta_v)
        mv = meta_v[t, pl.ds(0, 16)]
        P = [mv[i] for i in range(NSUB + 1)]

        def sub_body(l, carry):
            ch = l // NSUB
            s = l % NSUB
            p0 = P[0]
            p1 = P[1]
            for q in range(1, NSUB):
                p0 = jnp.where(s >= q, P[q], p0)
                p1 = jnp.where(s >= q, P[q + 1], p1)
            base = t * NPT + s * SR
            if cheby:
                pltpu.sync_copy(aux_hbm.at[pl.ds(base, SR), ch, :], acc_v)
            else:
                pltpu.sync_copy(aux_hbm.at[0, pl.ds(0, SR), :], acc_v)
            g0 = (p0 // 128) * 128
            npass = (p1 - g0 + (ESTG - 1)) // ESTG

            def pass_body(pi, c2):
                gs = g0 + pi * ESTG
                pltpu.sync_copy(gidx_hbm.at[ch, pl.ds(gs, ESTG)], gidx_v)
                pltpu.sync_copy(sidx_hbm.at[pl.ds(gs, ESTG)], sidx_v)
                pltpu.sync_copy(w_hbm.at[pl.ds(gs, ESTG)], w_v)
                cnt = jnp.clip(p1 - gs, 0, ESTG)
                nblk = (cnt + (EBLK - 1)) // EBLK

                def blk_body(kb, c3):
                    off0 = kb * EBLK
                    pltpu.sync_copy(x_hbm.at[gidx_v.at[pl.ds(off0, EBLK)]],
                                    rows_v)
                    for mb in range(EBLK // 16):
                        off = off0 + mb * 16
                        d16 = sidx_v[pl.ds(off, 16)]
                        w16 = w_v[pl.ds(off, 16)]
                        e16 = gs + off + lax.broadcasted_iota(jnp.int32, (16,), 0)
                        m16 = (e16 >= p0) & (e16 < p1)
                        wm = jnp.where(m16, w16, jnp.float32(0.0))
                        dr = jnp.clip(d16 - base, 0, SR - 1)
                        djs = [dr[j] for j in range(16)]
                        wjs = [wm[j] for j in range(16)]

                        def _acc(c, cacc):
                            o = c * 16
                            for j in range(16):
                                rj = mb * 16 + j
                                acc_v[djs[j], pl.ds(o, 16)] = (
                                    acc_v[djs[j], pl.ds(o, 16)]
                                    + wjs[j] * rows_v[rj, pl.ds(o, 16)])
                            return cacc
                        lax.fori_loop(0, nchunk, _acc, 0)
                    return c3

                lax.fori_loop(0, nblk, blk_body, 0)
                return c2

            lax.fori_loop(0, npass, pass_body, 0)
            pltpu.sync_copy(acc_v, y_hbm.at[pl.ds(base, SR), ch, :])
            return carry

        lax.fori_loop(0, nc * NSUB, sub_body, 0)

    return pl.kernel(
        body,
        out_type=jax.ShapeDtypeStruct((NP, nc, wc), jnp.float32),
        mesh=_sc_mesh(),
        scratch_types=[
            pltpu.VMEM((TILES, 16), jnp.int32),
            pltpu.VMEM((ESTG,), jnp.int32),
            pltpu.VMEM((ESTG,), jnp.int32),
            pltpu.VMEM((ESTG,), jnp.float32),
            pltpu.VMEM((EBLK, wc), jnp.float32),
            pltpu.VMEM((SR, wc), jnp.float32),
        ],
    )


@functools.cache
def _deg_kernel():
    def body(sidx_hbm, w_hbm, meta_hbm, zb_hbm, degw_hbm,
             meta_v, sidx_v, w_v, dsub_v):
        t = lax.axis_index("s") * 2 + lax.axis_index("c")
        pltpu.sync_copy(meta_hbm, meta_v)
        mv = meta_v[t, pl.ds(0, 16)]
        P = [mv[i] for i in range(NSUB + 1)]
        iota = lax.broadcasted_iota(jnp.int32, (16,), 0)

        def sub_body(s, carry):
            p0 = P[0]
            p1 = P[1]
            for q in range(1, NSUB):
                p0 = jnp.where(s >= q, P[q], p0)
                p1 = jnp.where(s >= q, P[q + 1], p1)
            base = t * NPT + s * SR
            pltpu.sync_copy(zb_hbm, dsub_v)
            g0 = (p0 // 128) * 128
            npass = (p1 - g0 + (ESTG - 1)) // ESTG

            def pass_body(pi, c2):
                gs = g0 + pi * ESTG
                pltpu.sync_copy(sidx_hbm.at[pl.ds(gs, ESTG)], sidx_v)
                pltpu.sync_copy(w_hbm.at[pl.ds(gs, ESTG)], w_v)
                cnt = jnp.clip(p1 - gs, 0, ESTG)
                ngrp = (cnt + 15) // 16

                def grp_body(g, c3):
                    off = g * 16
                    d16 = sidx_v[pl.ds(off, 16)]
                    w16 = w_v[pl.ds(off, 16)]
                    e16 = gs + off + iota
                    m16 = (e16 >= p0) & (e16 < p1)
                    wm = jnp.where(m16, w16, jnp.float32(0.0))
                    dr = jnp.clip(d16 - base, 0, SR - 1)
                    for j in range(16):
                        dj = dr[j]
                        dsub_v[dj, pl.ds(0, 16)] = (
                            dsub_v[dj, pl.ds(0, 16)] + wm[j])
                    return c3

                lax.fori_loop(0, ngrp, grp_body, 0)
                return c2

            lax.fori_loop(0, npass, pass_body, 0)
            pltpu.sync_copy(dsub_v, degw_hbm.at[pl.ds(base, SR), :])
            return carry

        lax.fori_loop(0, NSUB, sub_body, 0)

    return pl.kernel(
        body,
        out_type=jax.ShapeDtypeStruct((NP, 16), jnp.float32),
        mesh=_sc_mesh(),
        scratch_types=[
            pltpu.VMEM((TILES, 16), jnp.int32),
            pltpu.VMEM((ESTG,), jnp.int32),
            pltpu.VMEM((ESTG,), jnp.float32),
            pltpu.VMEM((SR, 16), jnp.float32),
        ],
    )


def _rdeg_compact(degw_in, degw_out):
    def body(a_ref, b_ref, oa_ref, ob_ref):
        for src, dstr in ((a_ref, oa_ref), (b_ref, ob_ref)):
            d = src[...][:, 0:1]
            dstr[...] = jnp.where(d > 0, 1.0 / d, 0.0)

    spec = pl.BlockSpec((NBLK, 16), lambda i: (i, 0))
    ospec = pl.BlockSpec((NBLK, 1), lambda i: (i, 0))
    return pl.pallas_call(
        body,
        grid=(NP // NBLK,),
        in_specs=[spec, spec],
        out_specs=[ospec, ospec],
        out_shape=[jax.ShapeDtypeStruct((NP, 1), jnp.float32)] * 2,
    )(degw_in, degw_out)


def _scale_call(x, rdegs, isz):
    n = len(rdegs)

    def body(*refs):
        x_ref = refs[0]
        xv = x_ref[...]
        for k in range(n):
            refs[1 + n + k][...] = xv * refs[1 + k][...][:, :, None]

    xspec = pl.BlockSpec((NBLK, B, isz), lambda i: (i, 0, 0))
    rspec = pl.BlockSpec((NBLK, 1), lambda i: (i, 0))
    out = pl.pallas_call(
        body,
        grid=(NP // NBLK,),
        in_specs=[xspec] + [rspec] * n,
        out_specs=[xspec] * n,
        out_shape=[jax.ShapeDtypeStruct((NP, B, isz), jnp.float32)] * n,
    )(x, *rdegs)
    return out


def _gate_call(xs, Wm, bg, in_dim, isz):
    def body(x0, x1, x2, x3, x4, w_ref, b_ref, u_ref, xc0_ref):
        flat0 = x0[...].reshape(NBLK * B, isz)
        acc = jnp.dot(flat0, w_ref[0], preferred_element_type=jnp.float32)
        for m, xr in enumerate((x1, x2, x3, x4)):
            acc = acc + jnp.dot(xr[...].reshape(NBLK * B, isz), w_ref[m + 1],
                                preferred_element_type=jnp.float32)
        val = jax.nn.sigmoid(acc + b_ref[...])
        r = val[:, :RNN]
        u = val[:, RNN:]
        hx = flat0[:, in_dim:in_dim + RNN]
        u_ref[...] = u.reshape(NBLK, B, RNN)
        parts = [flat0[:, :in_dim], r * hx]
        if isz > in_dim + RNN:
            parts.append(jnp.zeros((NBLK * B, isz - in_dim - RNN), jnp.float32))
        xc0_ref[...] = jnp.concatenate(parts, axis=1).reshape(NBLK, B, isz)

    xspec = pl.BlockSpec((NBLK, B, isz), lambda i: (i, 0, 0))
    return pl.pallas_call(
        body,
        grid=(NP // NBLK,),
        in_specs=[xspec] * 5 + [
            pl.BlockSpec((5, isz, 2 * RNN), lambda i: (0, 0, 0)),
            pl.BlockSpec((2 * RNN,), lambda i: (0,)),
        ],
        out_specs=[
            pl.BlockSpec((NBLK, B, RNN), lambda i: (i, 0, 0)),
            xspec,
        ],
        out_shape=[
            jax.ShapeDtypeStruct((NP, B, RNN), jnp.float32),
            jax.ShapeDtypeStruct((NP, B, isz), jnp.float32),
        ],
    )(*xs, Wm, bg)


def _cand_call(x0g, xs, Wm, bc, u_arr, in_dim, isz, extra, wp=None, bpv=None):
    last = extra is None

    def body(x0g_ref, x0, x1, x2, x3, x4, w_ref, b_ref, u_ref, *rest):
        if last:
            wp_ref, bp_ref, newh_ref, proj_ref = rest
        else:
            hx1_ref, newh_ref, x0g1_ref = rest
        flat0 = x0[...].reshape(NBLK * B, isz)
        acc = jnp.dot(flat0, w_ref[0], preferred_element_type=jnp.float32)
        for m, xr in enumerate((x1, x2, x3, x4)):
            acc = acc + jnp.dot(xr[...].reshape(NBLK * B, isz), w_ref[m + 1],
                                preferred_element_type=jnp.float32)
        c = jnp.tanh(acc + b_ref[...])
        hx = x0g_ref[...].reshape(NBLK * B, isz)[:, in_dim:in_dim + RNN]
        u = u_ref[...].reshape(NBLK * B, RNN)
        newh = u * hx + (1.0 - u) * c
        newh_ref[...] = newh.reshape(NBLK, B, RNN)
        if last:
            pr = jnp.dot(newh, wp_ref[...],
                         preferred_element_type=jnp.float32) + bp_ref[...]
            proj_ref[...] = pr.reshape(NBLK, B, 1)
        else:
            x0g1_ref[...] = jnp.concatenate(
                [newh, hx1_ref[...].reshape(NBLK * B, RNN)],
                axis=1).reshape(NBLK, B, 2 * RNN)

    xspec = pl.BlockSpec((NBLK, B, isz), lambda i: (i, 0, 0))
    hspec = pl.BlockSpec((NBLK, B, RNN), lambda i: (i, 0, 0))
    in_specs = [xspec] * 6 + [
        pl.BlockSpec((5, isz, RNN), lambda i: (0, 0, 0)),
        pl.BlockSpec((RNN,), lambda i: (0,)),
        hspec,
    ]
    inputs = [x0g, *xs, Wm, bc, u_arr]
    if last:
        in_specs += [pl.BlockSpec((RNN, 1), lambda i: (0, 0)),
                     pl.BlockSpec((1,), lambda i: (0,))]
        inputs += [wp, bpv]
        out_specs = [hspec, pl.BlockSpec((NBLK, B, 1), lambda i: (i, 0, 0))]
        out_shape = [jax.ShapeDtypeStruct((NP, B, RNN), jnp.float32),
                     jax.ShapeDtypeStruct((NP, B, 1), jnp.float32)]
    else:
        in_specs += [hspec]
        inputs += [extra]
        out_specs = [hspec, pl.BlockSpec((NBLK, B, 2 * RNN), lambda i: (i, 0, 0))]
        out_shape = [jax.ShapeDtypeStruct((NP, B, RNN), jnp.float32),
                     jax.ShapeDtypeStruct((NP, B, 2 * RNN), jnp.float32)]

    return pl.pallas_call(
        body,
        grid=(NP // NBLK,),
        in_specs=in_specs,
        out_specs=out_specs,
        out_shape=out_shape,
    )(*inputs)


def _wmat(W, isz, isz_pad):
    Wm = W.reshape(isz, 5, -1).transpose(1, 0, 2)
    sign = jnp.array([1.0, 1.0, -1.0, 1.0, -1.0], jnp.float32)[:, None, None]
    return jnp.pad(Wm * sign, ((0, 0), (0, isz_pad - isz), (0, 0)))


def kernel(inputs, hidden_state, edge_index, edge_weight,
           Wg0, bg0, Wc0, bc0, Wg1, bg1, Wc1, bc1, Wp, bp):
    src = edge_index[0]
    dst = edge_index[1]
    ew = edge_weight

    d_dst, d_src, d_ew = lax.sort([dst, src, ew], num_keys=1)
    s_src, s_dst, s_ew = lax.sort([src, dst, ew], num_keys=1)
    bounds = jnp.arange(0, NP + 1, SR, dtype=jnp.int32)
    P1 = jnp.searchsorted(d_dst, bounds).astype(jnp.int32)
    P2 = jnp.searchsorted(s_src, bounds).astype(jnp.int32)
    midx = jnp.minimum(NSUB * jnp.arange(TILES, dtype=jnp.int32)[:, None]
                       + jnp.arange(16, dtype=jnp.int32)[None, :],
                       NSUB * TILES)
    meta1 = P1[midx]
    meta2 = P2[midx]

    pad = EP - E
    zi = jnp.zeros((pad,), jnp.int32)
    zf = jnp.zeros((pad,), jnp.float32)
    d_src_p = jnp.concatenate([d_src, zi])
    d_dst_p = jnp.concatenate([d_dst, zi])
    d_ew_p = jnp.concatenate([d_ew, zf])
    s_src_p = jnp.concatenate([s_src, zi])
    s_dst_p = jnp.concatenate([s_dst, zi])
    s_ew_p = jnp.concatenate([s_ew, zf])

    gidx1 = {1: d_src_p[None], 2: jnp.stack([2 * d_src_p, 2 * d_src_p + 1])}
    gidx2 = {1: s_dst_p[None], 2: jnp.stack([2 * s_dst_p, 2 * s_dst_p + 1])}

    zb16 = jnp.zeros((SR, 16), jnp.float32)
    deg = _deg_kernel()
    degw_in = deg(d_dst_p, d_ew_p, meta1, zb16)
    degw_out = deg(s_src_p, s_ew_p, meta2, zb16)
    rdeg_in, rdeg_out = _rdeg_compact(degw_in, degw_out)
    d_ewm2_p = jnp.float32(-2.0) * d_ew_p
    s_ewm2_p = jnp.float32(-2.0) * s_ew_p

    hx0 = jnp.pad(jnp.moveaxis(hidden_state[0].reshape(B, N, RNN), 0, 1),
                  ((0, NP - N), (0, 0), (0, 0)))
    hx1 = jnp.pad(jnp.moveaxis(hidden_state[1].reshape(B, N, RNN), 0, 1),
                  ((0, NP - N), (0, 0), (0, 0)))
    inp0 = jnp.pad(jnp.moveaxis(inputs.reshape(B, N, 1), 0, 1),
                   ((0, NP - N), (0, 0), (0, 0)))
    x0g0 = jnp.concatenate(
        [inp0, hx0, jnp.zeros((NP, B, 7), jnp.float32)], axis=2)

    zb = {640: jnp.zeros((1, SR, 640), jnp.float32),
          512: jnp.zeros((1, SR, 512), jnp.float32)}

    def gconv_set(x0, nc, wc):
        isz = (wc * nc) // B
        sp = _spmm_kernel(nc, wc, False)
        sc = _spmm_kernel(nc, wc, True)
        x0a = x0.reshape(NP, nc, wc)
        (x0s,) = _scale_call(x0, [rdeg_out], isz)
        A = sp(x0s.reshape(NP * nc, wc), zb[wc], gidx1[nc], d_dst_p,
               d_ew_p, meta1)
        Ab = A.reshape(NP, B, isz)
        A1, A2 = _scale_call(Ab, [rdeg_out, rdeg_in], isz)
        t2 = sc(A1.reshape(NP * nc, wc), x0a, gidx1[nc], d_dst_p,
                d_ewm2_p, meta1)
        C = sp(A2.reshape(NP * nc, wc), zb[wc], gidx2[nc], s_src_p,
               s_ew_p, meta2)
        (C2,) = _scale_call(C.reshape(NP, B, isz), [rdeg_in], isz)
        t4 = sc(C2.reshape(NP * nc, wc), A, gidx2[nc], s_src_p,
                s_ewm2_p, meta2)
        shp = (NP, B, isz)
        return [x0, Ab, t2.reshape(shp), C.reshape(shp), t4.reshape(shp)]

    xs = gconv_set(x0g0, 1, 640)
    u0, xc0 = _gate_call(xs, _wmat(Wg0, 33, 40), bg0, 1, 40)
    xsc = gconv_set(xc0, 1, 640)
    newh0, x0g1 = _cand_call(x0g0, xsc, _wmat(Wc0, 33, 40), bc0, u0, 1, 40,
                             hx1)

    xs = gconv_set(x0g1, 2, 512)
    u1, xc1 = _gate_call(xs, _wmat(Wg1, 64, 64), bg1, RNN, 64)
    xsc = gconv_set(xc1, 2, 512)
    newh1, proj = _cand_call(x0g1, xsc, _wmat(Wc1, 64, 64), bc1, u1, RNN, 64,
                             None, Wp, bp)

    out0 = jnp.moveaxis(proj[:N, :, 0], 0, 1).reshape(B, N)
    h0 = jnp.moveaxis(newh0[:N], 0, 1).reshape(B, N * RNN)
    h1 = jnp.moveaxis(newh1[:N], 0, 1).reshape(B, N * RNN)
    return (out0, jnp.stack([h0, h1]))


# speedup vs baseline: 2.4854x; 2.1751x over previous
import functools

import jax
import jax.numpy as jnp
from jax import lax
from jax.experimental import pallas as pl
from jax.experimental.pallas import tpu as pltpu
from jax.experimental.pallas import tpu_sc as plsc

N = 10000
NP = 10240
E = 160000
EP = 163840
B = 16
RNN = 32
TILES = 32
NPT = NP // TILES
SR = 64
NSUB = NPT // SR
ESTG = 1024
EBLK = 128
NBLK = 256


def _sc_mesh():
    return plsc.VectorSubcoreMesh(core_axis_name="c", subcore_axis_name="s")


@functools.cache
def _spmm_kernel(nc, wc, cheby):
    nchunk = wc // 16

    def body(x_hbm, aux_hbm, gidx_hbm, sidx_hbm, w_hbm, meta_hbm, y_hbm,
             meta_v, gidx_v, sidx_v, w_v, rows_v, acc_v):
        t = lax.axis_index("s") * 2 + lax.axis_index("c")
        pltpu.sync_copy(meta_hbm, meta_v)
        mv = meta_v[t, pl.ds(0, 16)]
        P = [mv[i] for i in range(NSUB + 1)]

        def sub_body(l, carry):
            ch = l // NSUB
            s = l % NSUB
            p0 = P[0]
            p1 = P[1]
            for q in range(1, NSUB):
                p0 = jnp.where(s >= q, P[q], p0)
                p1 = jnp.where(s >= q, P[q + 1], p1)
            base = t * NPT + s * SR
            if cheby:
                pltpu.sync_copy(aux_hbm.at[pl.ds(base, SR), ch, :], acc_v)
            else:
                pltpu.sync_copy(aux_hbm.at[0, pl.ds(0, SR), :], acc_v)
            g0 = (p0 // 128) * 128
            npass = (p1 - g0 + (ESTG - 1)) // ESTG

            def pass_body(pi, c2):
                gs = g0 + pi * ESTG
                pltpu.sync_copy(gidx_hbm.at[ch, pl.ds(gs, ESTG)], gidx_v)
                pltpu.sync_copy(sidx_hbm.at[pl.ds(gs, ESTG)], sidx_v)
                pltpu.sync_copy(w_hbm.at[pl.ds(gs, ESTG)], w_v)
                cnt = jnp.clip(p1 - gs, 0, ESTG)
                nblk = (cnt + (EBLK - 1)) // EBLK

                def blk_body(kb, c3):
                    off0 = kb * EBLK
                    pltpu.sync_copy(x_hbm.at[gidx_v.at[pl.ds(off0, EBLK)]],
                                    rows_v)
                    for mb in range(EBLK // 16):
                        off = off0 + mb * 16
                        d16 = sidx_v[pl.ds(off, 16)]
                        w16 = w_v[pl.ds(off, 16)]
                        e16 = gs + off + lax.broadcasted_iota(jnp.int32, (16,), 0)
                        m16 = (e16 >= p0) & (e16 < p1)
                        wm = jnp.where(m16, w16, jnp.float32(0.0))
                        dr = jnp.clip(d16 - base, 0, SR - 1)
                        djs = [dr[j] for j in range(16)]
                        wjs = [wm[j] for j in range(16)]

                        @plsc.parallel_loop(0, nchunk)
                        def _acc(c):
                            o = c * 16
                            for j in range(16):
                                rj = mb * 16 + j
                                acc_v[djs[j], pl.ds(o, 16)] = (
                                    acc_v[djs[j], pl.ds(o, 16)]
                                    + wjs[j] * rows_v[rj, pl.ds(o, 16)])
                    return c3

                lax.fori_loop(0, nblk, blk_body, 0)
                return c2

            lax.fori_loop(0, npass, pass_body, 0)
            pltpu.sync_copy(acc_v, y_hbm.at[pl.ds(base, SR), ch, :])
            return carry

        lax.fori_loop(0, nc * NSUB, sub_body, 0)

    return pl.kernel(
        body,
        out_type=jax.ShapeDtypeStruct((NP, nc, wc), jnp.float32),
        mesh=_sc_mesh(),
        scratch_types=[
            pltpu.VMEM((TILES, 16), jnp.int32),
            pltpu.VMEM((ESTG,), jnp.int32),
            pltpu.VMEM((ESTG,), jnp.int32),
            pltpu.VMEM((ESTG,), jnp.float32),
            pltpu.VMEM((EBLK, wc), jnp.float32),
            pltpu.VMEM((SR, wc), jnp.float32),
        ],
    )


@functools.cache
def _deg_kernel():
    def body(sidx_hbm, w_hbm, meta_hbm, zb_hbm, degw_hbm,
             meta_v, sidx_v, w_v, dsub_v):
        t = lax.axis_index("s") * 2 + lax.axis_index("c")
        pltpu.sync_copy(meta_hbm, meta_v)
        mv = meta_v[t, pl.ds(0, 16)]
        P = [mv[i] for i in range(NSUB + 1)]
        iota = lax.broadcasted_iota(jnp.int32, (16,), 0)

        def sub_body(s, carry):
            p0 = P[0]
            p1 = P[1]
            for q in range(1, NSUB):
                p0 = jnp.where(s >= q, P[q], p0)
                p1 = jnp.where(s >= q, P[q + 1], p1)
            base = t * NPT + s * SR
            pltpu.sync_copy(zb_hbm, dsub_v)
            g0 = (p0 // 128) * 128
            npass = (p1 - g0 + (ESTG - 1)) // ESTG

            def pass_body(pi, c2):
                gs = g0 + pi * ESTG
                pltpu.sync_copy(sidx_hbm.at[pl.ds(gs, ESTG)], sidx_v)
                pltpu.sync_copy(w_hbm.at[pl.ds(gs, ESTG)], w_v)
                cnt = jnp.clip(p1 - gs, 0, ESTG)
                ngrp = (cnt + 15) // 16

                def grp_body(g, c3):
                    off = g * 16
                    d16 = sidx_v[pl.ds(off, 16)]
                    w16 = w_v[pl.ds(off, 16)]
                    e16 = gs + off + iota
                    m16 = (e16 >= p0) & (e16 < p1)
                    wm = jnp.where(m16, w16, jnp.float32(0.0))
                    dr = jnp.clip(d16 - base, 0, SR - 1)
                    for j in range(16):
                        dj = dr[j]
                        dsub_v[dj, pl.ds(0, 16)] = (
                            dsub_v[dj, pl.ds(0, 16)] + wm[j])
                    return c3

                lax.fori_loop(0, ngrp, grp_body, 0)
                return c2

            lax.fori_loop(0, npass, pass_body, 0)
            pltpu.sync_copy(dsub_v, degw_hbm.at[pl.ds(base, SR), :])
            return carry

        lax.fori_loop(0, NSUB, sub_body, 0)

    return pl.kernel(
        body,
        out_type=jax.ShapeDtypeStruct((NP, 16), jnp.float32),
        mesh=_sc_mesh(),
        scratch_types=[
            pltpu.VMEM((TILES, 16), jnp.int32),
            pltpu.VMEM((ESTG,), jnp.int32),
            pltpu.VMEM((ESTG,), jnp.float32),
            pltpu.VMEM((SR, 16), jnp.float32),
        ],
    )


def _rdeg_compact(degw_in, degw_out):
    def body(a_ref, b_ref, oa_ref, ob_ref):
        for src, dstr in ((a_ref, oa_ref), (b_ref, ob_ref)):
            d = src[...][:, 0:1]
            dstr[...] = jnp.where(d > 0, 1.0 / d, 0.0)

    spec = pl.BlockSpec((NBLK, 16), lambda i: (i, 0))
    ospec = pl.BlockSpec((NBLK, 1), lambda i: (i, 0))
    return pl.pallas_call(
        body,
        grid=(NP // NBLK,),
        in_specs=[spec, spec],
        out_specs=[ospec, ospec],
        out_shape=[jax.ShapeDtypeStruct((NP, 1), jnp.float32)] * 2,
    )(degw_in, degw_out)


def _scale_call(x, rdegs, isz):
    n = len(rdegs)

    def body(*refs):
        x_ref = refs[0]
        xv = x_ref[...]
        for k in range(n):
            refs[1 + n + k][...] = xv * refs[1 + k][...][:, :, None]

    xspec = pl.BlockSpec((NBLK, B, isz), lambda i: (i, 0, 0))
    rspec = pl.BlockSpec((NBLK, 1), lambda i: (i, 0))
    out = pl.pallas_call(
        body,
        grid=(NP // NBLK,),
        in_specs=[xspec] + [rspec] * n,
        out_specs=[xspec] * n,
        out_shape=[jax.ShapeDtypeStruct((NP, B, isz), jnp.float32)] * n,
    )(x, *rdegs)
    return out


def _gate_call(xs, Wm, bg, in_dim, isz):
    def body(x0, x1, x2, x3, x4, w_ref, b_ref, u_ref, xc0_ref):
        flat0 = x0[...].reshape(NBLK * B, isz)
        acc = jnp.dot(flat0, w_ref[0], preferred_element_type=jnp.float32)
        for m, xr in enumerate((x1, x2, x3, x4)):
            acc = acc + jnp.dot(xr[...].reshape(NBLK * B, isz), w_ref[m + 1],
                                preferred_element_type=jnp.float32)
        val = jax.nn.sigmoid(acc + b_ref[...])
        r = val[:, :RNN]
        u = val[:, RNN:]
        hx = flat0[:, in_dim:in_dim + RNN]
        u_ref[...] = u.reshape(NBLK, B, RNN)
        parts = [flat0[:, :in_dim], r * hx]
        if isz > in_dim + RNN:
            parts.append(jnp.zeros((NBLK * B, isz - in_dim - RNN), jnp.float32))
        xc0_ref[...] = jnp.concatenate(parts, axis=1).reshape(NBLK, B, isz)

    xspec = pl.BlockSpec((NBLK, B, isz), lambda i: (i, 0, 0))
    return pl.pallas_call(
        body,
        grid=(NP // NBLK,),
        in_specs=[xspec] * 5 + [
            pl.BlockSpec((5, isz, 2 * RNN), lambda i: (0, 0, 0)),
            pl.BlockSpec((2 * RNN,), lambda i: (0,)),
        ],
        out_specs=[
            pl.BlockSpec((NBLK, B, RNN), lambda i: (i, 0, 0)),
            xspec,
        ],
        out_shape=[
            jax.ShapeDtypeStruct((NP, B, RNN), jnp.float32),
            jax.ShapeDtypeStruct((NP, B, isz), jnp.float32),
        ],
    )(*xs, Wm, bg)


def _cand_call(x0g, xs, Wm, bc, u_arr, in_dim, isz, extra, wp=None, bpv=None):
    last = extra is None

    def body(x0g_ref, x0, x1, x2, x3, x4, w_ref, b_ref, u_ref, *rest):
        if last:
            wp_ref, bp_ref, newh_ref, proj_ref = rest
        else:
            hx1_ref, newh_ref, x0g1_ref = rest
        flat0 = x0[...].reshape(NBLK * B, isz)
        acc = jnp.dot(flat0, w_ref[0], preferred_element_type=jnp.float32)
        for m, xr in enumerate((x1, x2, x3, x4)):
            acc = acc + jnp.dot(xr[...].reshape(NBLK * B, isz), w_ref[m + 1],
                                preferred_element_type=jnp.float32)
        c = jnp.tanh(acc + b_ref[...])
        hx = x0g_ref[...].reshape(NBLK * B, isz)[:, in_dim:in_dim + RNN]
        u = u_ref[...].reshape(NBLK * B, RNN)
        newh = u * hx + (1.0 - u) * c
        newh_ref[...] = newh.reshape(NBLK, B, RNN)
        if last:
            pr = jnp.dot(newh, wp_ref[...],
                         preferred_element_type=jnp.float32) + bp_ref[...]
            proj_ref[...] = pr.reshape(NBLK, B, 1)
        else:
            x0g1_ref[...] = jnp.concatenate(
                [newh, hx1_ref[...].reshape(NBLK * B, RNN)],
                axis=1).reshape(NBLK, B, 2 * RNN)

    xspec = pl.BlockSpec((NBLK, B, isz), lambda i: (i, 0, 0))
    hspec = pl.BlockSpec((NBLK, B, RNN), lambda i: (i, 0, 0))
    in_specs = [xspec] * 6 + [
        pl.BlockSpec((5, isz, RNN), lambda i: (0, 0, 0)),
        pl.BlockSpec((RNN,), lambda i: (0,)),
        hspec,
    ]
    inputs = [x0g, *xs, Wm, bc, u_arr]
    if last:
        in_specs += [pl.BlockSpec((RNN, 1), lambda i: (0, 0)),
                     pl.BlockSpec((1,), lambda i: (0,))]
        inputs += [wp, bpv]
        out_specs = [hspec, pl.BlockSpec((NBLK, B, 1), lambda i: (i, 0, 0))]
        out_shape = [jax.ShapeDtypeStruct((NP, B, RNN), jnp.float32),
                     jax.ShapeDtypeStruct((NP, B, 1), jnp.float32)]
    else:
        in_specs += [hspec]
        inputs += [extra]
        out_specs = [hspec, pl.BlockSpec((NBLK, B, 2 * RNN), lambda i: (i, 0, 0))]
        out_shape = [jax.ShapeDtypeStruct((NP, B, RNN), jnp.float32),
                     jax.ShapeDtypeStruct((NP, B, 2 * RNN), jnp.float32)]

    return pl.pallas_call(
        body,
        grid=(NP // NBLK,),
        in_specs=in_specs,
        out_specs=out_specs,
        out_shape=out_shape,
    )(*inputs)


def _wmat(W, isz, isz_pad):
    Wm = W.reshape(isz, 5, -1).transpose(1, 0, 2)
    sign = jnp.array([1.0, 1.0, -1.0, 1.0, -1.0], jnp.float32)[:, None, None]
    return jnp.pad(Wm * sign, ((0, 0), (0, isz_pad - isz), (0, 0)))


def kernel(inputs, hidden_state, edge_index, edge_weight,
           Wg0, bg0, Wc0, bc0, Wg1, bg1, Wc1, bc1, Wp, bp):
    src = edge_index[0]
    dst = edge_index[1]
    ew = edge_weight

    d_dst, d_src, d_ew = lax.sort([dst, src, ew], num_keys=1)
    s_src, s_dst, s_ew = lax.sort([src, dst, ew], num_keys=1)
    bounds = jnp.arange(0, NP + 1, SR, dtype=jnp.int32)
    P1 = jnp.searchsorted(d_dst, bounds).astype(jnp.int32)
    P2 = jnp.searchsorted(s_src, bounds).astype(jnp.int32)
    midx = jnp.minimum(NSUB * jnp.arange(TILES, dtype=jnp.int32)[:, None]
                       + jnp.arange(16, dtype=jnp.int32)[None, :],
                       NSUB * TILES)
    meta1 = P1[midx]
    meta2 = P2[midx]

    pad = EP - E
    zi = jnp.zeros((pad,), jnp.int32)
    zf = jnp.zeros((pad,), jnp.float32)
    d_src_p = jnp.concatenate([d_src, zi])
    d_dst_p = jnp.concatenate([d_dst, zi])
    d_ew_p = jnp.concatenate([d_ew, zf])
    s_src_p = jnp.concatenate([s_src, zi])
    s_dst_p = jnp.concatenate([s_dst, zi])
    s_ew_p = jnp.concatenate([s_ew, zf])

    gidx1 = {1: d_src_p[None], 2: jnp.stack([2 * d_src_p, 2 * d_src_p + 1])}
    gidx2 = {1: s_dst_p[None], 2: jnp.stack([2 * s_dst_p, 2 * s_dst_p + 1])}

    zb16 = jnp.zeros((SR, 16), jnp.float32)
    deg = _deg_kernel()
    degw_in = deg(d_dst_p, d_ew_p, meta1, zb16)
    degw_out = deg(s_src_p, s_ew_p, meta2, zb16)
    rdeg_in, rdeg_out = _rdeg_compact(degw_in, degw_out)
    d_ewm2_p = jnp.float32(-2.0) * d_ew_p
    s_ewm2_p = jnp.float32(-2.0) * s_ew_p

    hx0 = jnp.pad(jnp.moveaxis(hidden_state[0].reshape(B, N, RNN), 0, 1),
                  ((0, NP - N), (0, 0), (0, 0)))
    hx1 = jnp.pad(jnp.moveaxis(hidden_state[1].reshape(B, N, RNN), 0, 1),
                  ((0, NP - N), (0, 0), (0, 0)))
    inp0 = jnp.pad(jnp.moveaxis(inputs.reshape(B, N, 1), 0, 1),
                   ((0, NP - N), (0, 0), (0, 0)))
    x0g0 = jnp.concatenate(
        [inp0, hx0, jnp.zeros((NP, B, 7), jnp.float32)], axis=2)

    zb = {640: jnp.zeros((1, SR, 640), jnp.float32),
          512: jnp.zeros((1, SR, 512), jnp.float32)}

    def gconv_set(x0, nc, wc):
        isz = (wc * nc) // B
        sp = _spmm_kernel(nc, wc, False)
        sc = _spmm_kernel(nc, wc, True)
        x0a = x0.reshape(NP, nc, wc)
        (x0s,) = _scale_call(x0, [rdeg_out], isz)
        A = sp(x0s.reshape(NP * nc, wc), zb[wc], gidx1[nc], d_dst_p,
               d_ew_p, meta1)
        Ab = A.reshape(NP, B, isz)
        A1, A2 = _scale_call(Ab, [rdeg_out, rdeg_in], isz)
        t2 = sc(A1.reshape(NP * nc, wc), x0a, gidx1[nc], d_dst_p,
                d_ewm2_p, meta1)
        C = sp(A2.reshape(NP * nc, wc), zb[wc], gidx2[nc], s_src_p,
               s_ew_p, meta2)
        (C2,) = _scale_call(C.reshape(NP, B, isz), [rdeg_in], isz)
        t4 = sc(C2.reshape(NP * nc, wc), A, gidx2[nc], s_src_p,
                s_ewm2_p, meta2)
        shp = (NP, B, isz)
        return [x0, Ab, t2.reshape(shp), C.reshape(shp), t4.reshape(shp)]

    xs = gconv_set(x0g0, 1, 640)
    u0, xc0 = _gate_call(xs, _wmat(Wg0, 33, 40), bg0, 1, 40)
    xsc = gconv_set(xc0, 1, 640)
    newh0, x0g1 = _cand_call(x0g0, xsc, _wmat(Wc0, 33, 40), bc0, u0, 1, 40,
                             hx1)

    xs = gconv_set(x0g1, 2, 512)
    u1, xc1 = _gate_call(xs, _wmat(Wg1, 64, 64), bg1, RNN, 64)
    xsc = gconv_set(xc1, 2, 512)
    newh1, proj = _cand_call(x0g1, xsc, _wmat(Wc1, 64, 64), bc1, u1, RNN, 64,
                             None, Wp, bp)

    out0 = jnp.moveaxis(proj[:N, :, 0], 0, 1).reshape(B, N)
    h0 = jnp.moveaxis(newh0[:N], 0, 1).reshape(B, N * RNN)
    h1 = jnp.moveaxis(newh1[:N], 0, 1).reshape(B, N * RNN)
    return (out0, jnp.stack([h0, h1]))
